# Initial kernel scaffold; baseline (speedup 1.0000x reference)
import jax
import jax.numpy as jnp
from jax.experimental import pallas as pl


def kernel(x, v, z, pos, batch, W1, b1, W2, b2):
    raise NotImplementedError("write your pallas kernel here")


# speedup vs baseline: 9.8294x; 9.8294x over previous
import functools

import jax
import jax.numpy as jnp
import numpy as np
from jax import lax
from jax.experimental import pallas as pl
from jax.experimental.pallas import tpu as pltpu
from jax.experimental.pallas import tpu_sc as plsc

_MASSES = np.array([1.0, 1.008, 4.002602, 6.94, 9.0121831, 10.81, 12.011, 14.007, 15.999, 18.998403163, 20.1797, 22.98976928, 24.305, 26.9815385, 28.085, 30.973761998, 32.06, 35.45, 39.948, 39.0983, 40.078, 44.955908, 47.867, 50.9415, 51.9961, 54.938044, 55.845, 58.933194, 58.6934, 63.546, 65.38, 69.723, 72.63, 74.921595, 78.971, 79.904, 83.798, 85.4678, 87.62, 88.90584, 91.224, 92.90637, 95.95, 97.90721, 101.07, 102.9055, 106.42, 107.8682, 112.414, 114.818, 118.71, 121.76, 127.6, 126.90447, 131.293, 132.90545196, 137.327, 138.90547, 140.116, 140.90766, 144.242, 144.91276, 150.36, 151.964, 157.25, 158.92535, 162.5, 164.93033, 167.259, 168.93422, 173.054, 174.9668, 178.49, 180.94788, 183.84, 186.207, 190.23, 192.217, 195.084, 196.966569, 200.592, 204.38, 207.2, 208.9804, 208.98243, 209.98715, 222.01758, 223.01974, 226.02541, 227.02775, 232.0377, 231.03588, 238.02891, 237.04817, 244.06421, 243.06138, 247.07035, 247.07031, 251.07959, 252.083, 257.09511, 258.09843, 259.101, 262.11, 267.122, 268.126, 271.134, 270.133, 269.1338, 278.156, 281.165, 281.166, 285.177, 286.182, 289.19, 289.194, 293.204, 293.208, 294.214], dtype=np.float32)

S = 4096
N_ATOMS = 100000
NTILES = 32
CHUNK = 3128
NP = NTILES * CHUNK
GP = 3200
NGROUP = GP // 16
NJ = GP // 128
BA = 2048


def _mlp_body(x_ref, w1_ref, b1_ref, w2_ref, b2_ref, o_ref):
    xb = x_ref[...]
    h = lax.dot_general(xb, w1_ref[...], (((1,), (1,)), ((), ())),
                        preferred_element_type=jnp.float32)
    h = h + b1_ref[...]
    h = h * (1.0 / (1.0 + jnp.exp(-h)))
    o = lax.dot_general(w2_ref[...], h, (((1,), (1,)), ((), ())),
                        preferred_element_type=jnp.float32)
    o_ref[...] = o + b2_ref[...]


def _mlp(x, W1, b1, W2, b2):
    n, hdim = x.shape
    hh = W1.shape[0]
    grid = pl.cdiv(n, BA)
    return pl.pallas_call(
        _mlp_body,
        grid=(grid,),
        in_specs=[
            pl.BlockSpec((BA, hdim), lambda i: (i, 0)),
            pl.BlockSpec((hh, hdim), lambda i: (0, 0)),
            pl.BlockSpec((1, hh), lambda i: (0, 0)),
            pl.BlockSpec((1, hh), lambda i: (0, 0)),
            pl.BlockSpec((1, 1), lambda i: (0, 0)),
        ],
        out_specs=pl.BlockSpec((1, BA), lambda i: (0, i)),
        out_shape=jax.ShapeDtypeStruct((1, n), jnp.float32),
    )(x, W1, b1.reshape(1, -1), W2, b2.reshape(1, 1))


def _seg_body(out_hbm, z_hbm, posr_hbm, batch_hbm, masses_hbm, zseg_hbm,
              part_hbm,
              batch_v, z_v, out_v, posr_v, masses_v, vals_v, idx_v, acc_sh):
    c = lax.axis_index("c")
    s = lax.axis_index("s")
    wid = c * 16 + s
    base = wid * CHUNK
    count = jnp.minimum(jnp.int32(CHUNK), jnp.int32(N_ATOMS) - base)

    pltpu.sync_copy(masses_hbm, masses_v)
    pltpu.sync_copy(batch_hbm.at[pl.ds(base, CHUNK)], batch_v.at[pl.ds(0, CHUNK)])
    pltpu.sync_copy(z_hbm.at[pl.ds(base, CHUNK)], z_v.at[pl.ds(0, CHUNK)])
    pltpu.sync_copy(out_hbm.at[pl.ds(base, CHUNK)], out_v.at[pl.ds(0, CHUNK)])
    pltpu.sync_copy(posr_hbm.at[pl.ds(3 * base, 3 * CHUNK)],
                    posr_v.at[pl.ds(0, 3 * CHUNK)])
    pltpu.sync_copy(zseg_hbm.at[pl.ds(s * (S // 16), S // 16)],
                    acc_sh.at[pl.ds(s * (S // 16), S // 16)])

    iota = lax.iota(jnp.int32, 16)

    def grp(g, carry):
        a0 = g * 16
        av = a0 + iota
        valid = av < count
        b16 = batch_v[pl.ds(a0, 16)]
        z16 = z_v[pl.ds(a0, 16)] & 127
        o16 = out_v[pl.ds(a0, 16)]
        p0 = a0 * 3
        px = plsc.load_gather(posr_v, [p0 + iota * 3])
        py = plsc.load_gather(posr_v, [p0 + iota * 3 + 1])
        pz = plsc.load_gather(posr_v, [p0 + iota * 3 + 2])
        m16 = plsc.load_gather(masses_v, [z16])
        zf = jnp.zeros((16,), jnp.float32)
        o16 = jnp.where(valid, o16, zf)
        m16 = jnp.where(valid, m16, zf)
        px = jnp.where(valid, px, zf)
        py = jnp.where(valid, py, zf)
        pz = jnp.where(valid, pz, zf)
        bidx = jnp.where(valid, b16, 0)
        ch = (o16 * px, o16 * py, o16 * pz, o16,
              m16 * px, m16 * py, m16 * pz, m16)
        for k in range(8):
            plsc.store_scatter(vals_v, [av, jnp.full((16,), k, jnp.int32)],
                               ch[k])
        plsc.store_scatter(idx_v, [av >> 7, av & 127], bidx)
        return carry

    lax.fori_loop(0, NGROUP, grp, 0)

    plsc.subcore_barrier()
    for j in range(NJ):
        pltpu.sync_copy(vals_v.at[pl.ds(j * 128, 128)],
                        acc_sh.at[idx_v.at[j]], add=True)
    plsc.subcore_barrier()
    pltpu.sync_copy(acc_sh.at[pl.ds(s * (S // 16), S // 16)],
                    part_hbm.at[c, pl.ds(s * (S // 16), S // 16)])


@functools.cache
def _seg_kernel():
    return pl.kernel(
        _seg_body,
        out_type=jax.ShapeDtypeStruct((2, S, 8), jnp.float32),
        mesh=plsc.VectorSubcoreMesh(core_axis_name="c", subcore_axis_name="s",
                                    num_cores=2, num_subcores=16),
        compiler_params=pltpu.CompilerParams(needs_layout_passes=False,
                                             use_tc_tiling_on_sc=False),
        scratch_types=[
            pltpu.VMEM((GP,), jnp.int32),
            pltpu.VMEM((GP,), jnp.int32),
            pltpu.VMEM((GP,), jnp.float32),
            pltpu.VMEM((3 * GP,), jnp.float32),
            pltpu.VMEM((128,), jnp.float32),
            pltpu.VMEM((GP, 8), jnp.float32),
            pltpu.VMEM((NJ, 128), jnp.int32),
            pltpu.VMEM_SHARED((S, 8), jnp.float32),
        ],
    )


def _fin_body(p_ref, o_ref):
    p = p_ref[...]
    val = p[0] + p[1]
    a = val[:, 0:3]
    b = val[:, 3:4]
    num = val[:, 4:7]
    den = val[:, 7:8]
    den = jnp.where(den == 0.0, 1.0, den)
    red = a - (num / den) * b
    o_ref[...] = jnp.sqrt(jnp.sum(red * red, axis=1, keepdims=True))


def kernel(x, v, z, pos, batch, W1, b1, W2, b2):
    n = x.shape[0]
    out1 = _mlp(x, W1, b1, W2, b2).reshape(-1)
    pad = NP - n
    outp = jnp.pad(out1, (0, pad))
    zp = jnp.pad(z, (0, pad))
    batchp = jnp.pad(batch, (0, pad))
    posrp = jnp.pad(pos.reshape(-1), (0, 3 * pad))
    masses128 = jnp.asarray(np.pad(_MASSES, (0, 128 - _MASSES.shape[0])))
    zseg = jnp.zeros((S, 8), jnp.float32)
    partials = _seg_kernel()(outp, zp, posrp, batchp, masses128, zseg)
    return pl.pallas_call(
        _fin_body,
        out_shape=jax.ShapeDtypeStruct((S, 1), jnp.float32),
    )(partials)
